# baseline (speedup 1.0000x reference)
import functools

import jax
import jax.numpy as jnp
from jax import lax
from jax.experimental import pallas as pl
from jax.experimental.pallas import tpu as pltpu
from jax.experimental.pallas import tpu_sc as plsc

K = 1024
D = 32
BLOCK = 512


def _vq_argmin_kernel(x_ref, emb_ref, c_ref, l_ref):
    x = x_ref[...]
    e = emb_ref[...]
    e2 = jnp.sum(e * e, axis=1)
    x2 = jnp.sum(x * x, axis=1)
    xe2 = lax.dot_general(x * -2.0, e, (((1,), (1,)), ((), ())),
                          preferred_element_type=jnp.float32)
    d = (x2[:, None] + e2[None, :]) + xe2
    m = jnp.min(d, axis=1)
    iota = lax.broadcasted_iota(jnp.int32, (BLOCK, K), 1)
    c = jnp.min(jnp.where(d <= m[:, None], iota, K), axis=1).astype(jnp.int32)
    c_ref[...] = c.reshape(1, 1, BLOCK)
    l_ref[...] = (m * (0.2 / D)).reshape(1, 1, BLOCK)


def _argmin_stage(flat, emb):
    n = flat.shape[0]
    nb = n // BLOCK
    c, l = pl.pallas_call(
        _vq_argmin_kernel,
        grid=(nb,),
        in_specs=[
            pl.BlockSpec((BLOCK, D), lambda i: (i, 0)),
            pl.BlockSpec((K, D), lambda i: (0, 0)),
        ],
        out_specs=[
            pl.BlockSpec((1, 1, BLOCK), lambda i: (i, 0, 0)),
            pl.BlockSpec((1, 1, BLOCK), lambda i: (i, 0, 0)),
        ],
        out_shape=[
            jax.ShapeDtypeStruct((nb, 1, BLOCK), jnp.int32),
            jax.ShapeDtypeStruct((nb, 1, BLOCK), jnp.float32),
        ],
    )(flat, emb)
    return c.reshape(n), l.reshape(n)


CHUNK = 128


def _make_gather(n):
    info = plsc.get_sparse_core_info()
    nw = info.num_cores * info.num_subcores
    b_per_w = n // nw
    n_chunks = b_per_w // CHUNK
    mesh = plsc.VectorSubcoreMesh(core_axis_name="c", subcore_axis_name="s")

    @functools.partial(
        pl.kernel, mesh=mesh,
        out_type=jax.ShapeDtypeStruct((n, 128), jnp.float32),
        scratch_types=[
            pltpu.VMEM((n_chunks, CHUNK), jnp.int32),
            pltpu.VMEM((CHUNK, 128), jnp.float32),
            pltpu.VMEM((CHUNK, 128), jnp.float32),
            pltpu.SemaphoreType.DMA,
            pltpu.SemaphoreType.DMA,
            pltpu.SemaphoreType.DMA,
        ],
    )
    def gather(table_hbm, idx_hbm, out_hbm, idx_v, rows_a, rows_b, sem_g, sem_a, sem_b):
        wid = lax.axis_index("s") * info.num_cores + lax.axis_index("c")
        row0 = wid * n_chunks
        pltpu.sync_copy(idx_hbm.at[pl.ds(row0, n_chunks)], idx_v)
        bufs = (rows_a, rows_b)
        sems = (sem_a, sem_b)
        for j in range(n_chunks):
            buf, sem = bufs[j % 2], sems[j % 2]
            if j >= 2:
                pltpu.make_async_copy(
                    buf,
                    out_hbm.at[pl.ds((row0 + j - 2) * CHUNK, CHUNK)],
                    sem).wait()
            pltpu.async_copy(table_hbm.at[idx_v.at[j]], buf, sem_g).wait()
            pltpu.async_copy(buf,
                             out_hbm.at[pl.ds((row0 + j) * CHUNK, CHUNK)],
                             sem)
        for j in range(n_chunks - 2, n_chunks):
            pltpu.make_async_copy(
                bufs[j % 2],
                out_hbm.at[pl.ds((row0 + j) * CHUNK, CHUNK)],
                sems[j % 2]).wait()

    return gather


def kernel(h, emb):
    flat = h.reshape(-1, D)
    n = flat.shape[0]
    c, l = _argmin_stage(flat, emb)
    table = jnp.pad(emb, ((0, 0), (0, 128 - D)))
    q = _make_gather(n)(table, c.reshape(n // CHUNK, CHUNK))[:, :D]
    return q.reshape(h.shape), c.reshape(n, 1), l


# speedup vs baseline: 1.5052x; 1.5052x over previous
import functools

import jax
import jax.numpy as jnp
from jax import lax
from jax.experimental import pallas as pl
from jax.experimental.pallas import tpu as pltpu
from jax.experimental.pallas import tpu_sc as plsc

K = 1024
D = 32
BLOCK = 512


def _vq_argmin_kernel(xt_ref, emb_ref, e2_ref, x2_ref, c_ref, l_ref):
    xt = xt_ref[...]
    e = emb_ref[...]
    e2c = e2_ref[...]
    x2r = x2_ref[...].reshape(1, BLOCK)
    xe2 = lax.dot_general(e, xt * -2.0, (((1,), (0,)), ((), ())),
                          preferred_element_type=jnp.float32)
    d = (x2r + e2c) + xe2
    m = jnp.min(d, axis=0, keepdims=True)
    iota = lax.broadcasted_iota(jnp.int32, (K, BLOCK), 0)
    c = jnp.min(jnp.where(d <= m, iota, K), axis=0).astype(jnp.int32)
    c_ref[...] = c.reshape(1, 1, BLOCK)
    l_ref[...] = (m * (0.2 / D)).reshape(1, 1, BLOCK)


def _argmin_stage(flat, emb):
    n = flat.shape[0]
    nb = n // BLOCK
    xt = flat.T
    e2 = jnp.sum(emb ** 2, axis=-1)[:, None]
    x2 = jnp.sum(flat ** 2, axis=-1).reshape(nb, 1, BLOCK)
    c, l = pl.pallas_call(
        _vq_argmin_kernel,
        grid=(nb,),
        in_specs=[
            pl.BlockSpec((D, BLOCK), lambda i: (0, i)),
            pl.BlockSpec((K, D), lambda i: (0, 0)),
            pl.BlockSpec((K, 1), lambda i: (0, 0)),
            pl.BlockSpec((1, 1, BLOCK), lambda i: (i, 0, 0)),
        ],
        out_specs=[
            pl.BlockSpec((1, 1, BLOCK), lambda i: (i, 0, 0)),
            pl.BlockSpec((1, 1, BLOCK), lambda i: (i, 0, 0)),
        ],
        out_shape=[
            jax.ShapeDtypeStruct((nb, 1, BLOCK), jnp.int32),
            jax.ShapeDtypeStruct((nb, 1, BLOCK), jnp.float32),
        ],
    )(xt, emb, e2, x2)
    return c.reshape(n), l.reshape(n)


CHUNK = 128


def _make_gather(n):
    info = plsc.get_sparse_core_info()
    nw = info.num_cores * info.num_subcores
    b_per_w = n // nw
    n_chunks = b_per_w // CHUNK
    mesh = plsc.VectorSubcoreMesh(core_axis_name="c", subcore_axis_name="s")

    @functools.partial(
        pl.kernel, mesh=mesh,
        out_type=jax.ShapeDtypeStruct((n, 128), jnp.float32),
        scratch_types=[
            pltpu.VMEM((n_chunks, CHUNK), jnp.int32),
            pltpu.VMEM((CHUNK, 128), jnp.float32),
            pltpu.VMEM((CHUNK, 128), jnp.float32),
            pltpu.SemaphoreType.DMA,
            pltpu.SemaphoreType.DMA,
            pltpu.SemaphoreType.DMA,
        ],
    )
    def gather(table_hbm, idx_hbm, out_hbm, idx_v, rows_a, rows_b, sem_g, sem_a, sem_b):
        wid = lax.axis_index("s") * info.num_cores + lax.axis_index("c")
        row0 = wid * n_chunks
        pltpu.sync_copy(idx_hbm.at[pl.ds(row0, n_chunks)], idx_v)
        bufs = (rows_a, rows_b)
        sems = (sem_a, sem_b)
        for j in range(n_chunks):
            buf, sem = bufs[j % 2], sems[j % 2]
            if j >= 2:
                pltpu.make_async_copy(
                    buf,
                    out_hbm.at[pl.ds((row0 + j - 2) * CHUNK, CHUNK)],
                    sem).wait()
            pltpu.async_copy(table_hbm.at[idx_v.at[j]], buf, sem_g).wait()
            pltpu.async_copy(buf,
                             out_hbm.at[pl.ds((row0 + j) * CHUNK, CHUNK)],
                             sem)
        for j in range(n_chunks - 2, n_chunks):
            pltpu.make_async_copy(
                bufs[j % 2],
                out_hbm.at[pl.ds((row0 + j) * CHUNK, CHUNK)],
                sems[j % 2]).wait()

    return gather


def kernel(h, emb):
    flat = h.reshape(-1, D)
    n = flat.shape[0]
    c, l = _argmin_stage(flat, emb)
    table = jnp.pad(emb, ((0, 0), (0, 128 - D)))
    q = _make_gather(n)(table, c.reshape(n // CHUNK, CHUNK))[:, :D]
    return q.reshape(h.shape), c.reshape(n, 1), l


# speedup vs baseline: 1.5270x; 1.0144x over previous
import functools

import jax
import jax.numpy as jnp
from jax import lax
from jax.experimental import pallas as pl
from jax.experimental.pallas import tpu as pltpu
from jax.experimental.pallas import tpu_sc as plsc

K = 1024
D = 32
BLOCK = 512


def _vq_argmin_kernel(xt_ref, emb_ref, e2_ref, x2_ref, c_ref, l_ref, pad_ref):
    xt = xt_ref[...]
    e = emb_ref[...]
    e2c = e2_ref[...]
    x2r = x2_ref[...].reshape(1, BLOCK)
    xe2 = lax.dot_general(e, xt * -2.0, (((1,), (0,)), ((), ())),
                          preferred_element_type=jnp.float32)
    d = (x2r + e2c) + xe2
    m = jnp.min(d, axis=0, keepdims=True)
    iota = lax.broadcasted_iota(jnp.int32, (K, BLOCK), 0)
    c = jnp.min(jnp.where(d <= m, iota, K), axis=0).astype(jnp.int32)
    c_ref[...] = c.reshape(1, 1, BLOCK)
    l_ref[...] = (m * (0.2 / D)).reshape(1, 1, BLOCK)

    @pl.when(pl.program_id(0) == 0)
    def _():
        pad_ref[...] = jnp.pad(e, ((0, 0), (0, 128 - D)))


def _argmin_stage(flat, emb):
    n = flat.shape[0]
    nb = n // BLOCK
    xt = flat.T
    e2 = jnp.sum(emb ** 2, axis=-1)[:, None]
    x2 = jnp.sum(flat ** 2, axis=-1).reshape(nb, 1, BLOCK)
    c, l, table = pl.pallas_call(
        _vq_argmin_kernel,
        grid=(nb,),
        in_specs=[
            pl.BlockSpec((D, BLOCK), lambda i: (0, i)),
            pl.BlockSpec((K, D), lambda i: (0, 0)),
            pl.BlockSpec((K, 1), lambda i: (0, 0)),
            pl.BlockSpec((1, 1, BLOCK), lambda i: (i, 0, 0)),
        ],
        out_specs=[
            pl.BlockSpec((1, 1, BLOCK), lambda i: (i, 0, 0)),
            pl.BlockSpec((1, 1, BLOCK), lambda i: (i, 0, 0)),
            pl.BlockSpec((K, 128), lambda i: (0, 0)),
        ],
        out_shape=[
            jax.ShapeDtypeStruct((nb, 1, BLOCK), jnp.int32),
            jax.ShapeDtypeStruct((nb, 1, BLOCK), jnp.float32),
            jax.ShapeDtypeStruct((K, 128), jnp.float32),
        ],
        compiler_params=pltpu.CompilerParams(
            dimension_semantics=("arbitrary",),
        ),
    )(xt, emb, e2, x2)
    return c.reshape(n), l.reshape(n), table


CHUNK = 128


def _make_gather(n):
    info = plsc.get_sparse_core_info()
    nw = info.num_cores * info.num_subcores
    b_per_w = n // nw
    n_chunks = b_per_w // CHUNK
    mesh = plsc.VectorSubcoreMesh(core_axis_name="c", subcore_axis_name="s")

    @functools.partial(
        pl.kernel, mesh=mesh,
        out_type=jax.ShapeDtypeStruct((n, 128), jnp.float32),
        scratch_types=[
            pltpu.VMEM((n_chunks, CHUNK), jnp.int32),
            pltpu.VMEM((CHUNK, 128), jnp.float32),
            pltpu.VMEM((CHUNK, 128), jnp.float32),
            pltpu.SemaphoreType.DMA,
            pltpu.SemaphoreType.DMA,
            pltpu.SemaphoreType.DMA,
        ],
    )
    def gather(table_hbm, idx_hbm, out_hbm, idx_v, rows_a, rows_b, sem_g, sem_a, sem_b):
        wid = lax.axis_index("s") * info.num_cores + lax.axis_index("c")
        row0 = wid * n_chunks
        pltpu.sync_copy(idx_hbm.at[pl.ds(row0, n_chunks)], idx_v)
        bufs = (rows_a, rows_b)
        sems = (sem_a, sem_b)
        for j in range(n_chunks):
            buf, sem = bufs[j % 2], sems[j % 2]
            if j >= 2:
                pltpu.make_async_copy(
                    buf,
                    out_hbm.at[pl.ds((row0 + j - 2) * CHUNK, CHUNK)],
                    sem).wait()
            pltpu.async_copy(table_hbm.at[idx_v.at[j]], buf, sem_g).wait()
            pltpu.async_copy(buf,
                             out_hbm.at[pl.ds((row0 + j) * CHUNK, CHUNK)],
                             sem)
        for j in range(n_chunks - 2, n_chunks):
            pltpu.make_async_copy(
                bufs[j % 2],
                out_hbm.at[pl.ds((row0 + j) * CHUNK, CHUNK)],
                sems[j % 2]).wait()

    return gather


def kernel(h, emb):
    flat = h.reshape(-1, D)
    n = flat.shape[0]
    c, l, table = _argmin_stage(flat, emb)
    q = _make_gather(n)(table, c.reshape(n // CHUNK, CHUNK))[:, :D]
    return q.reshape(h.shape), c.reshape(n, 1), l


# speedup vs baseline: 1.7468x; 1.1440x over previous
import functools

import jax
import jax.numpy as jnp
from jax import lax
from jax.experimental import pallas as pl
from jax.experimental.pallas import tpu as pltpu
from jax.experimental.pallas import tpu_sc as plsc

K = 1024
D = 32
BLOCK = 1024


def _vq_argmin_kernel(xt_ref, emb_ref, e2_ref, x2_ref, c_ref, l_ref, pad_ref):
    xt = xt_ref[...]
    e = emb_ref[...]
    e2c = e2_ref[...]
    x2r = x2_ref[...].reshape(1, BLOCK)
    xe2 = lax.dot_general(e, xt * -2.0, (((1,), (0,)), ((), ())),
                          preferred_element_type=jnp.float32)
    d = (x2r + e2c) + xe2
    m = jnp.min(d, axis=0, keepdims=True)
    iota = lax.broadcasted_iota(jnp.int32, (K, BLOCK), 0)
    c = jnp.min(jnp.where(d <= m, iota, K), axis=0).astype(jnp.int32)
    c_ref[...] = c.reshape(1, 1, BLOCK)
    l_ref[...] = (m * (0.2 / D)).reshape(1, 1, BLOCK)

    @pl.when(pl.program_id(0) == 0)
    def _():
        pad_ref[...] = jnp.pad(e, ((0, 0), (0, 128 - D)))


def _argmin_stage(flat, emb):
    n = flat.shape[0]
    nb = n // BLOCK
    xt = flat.T
    e2 = jnp.sum(emb ** 2, axis=-1)[:, None]
    x2 = jnp.sum(flat ** 2, axis=-1).reshape(nb, 1, BLOCK)
    c, l, table = pl.pallas_call(
        _vq_argmin_kernel,
        grid=(nb,),
        in_specs=[
            pl.BlockSpec((D, BLOCK), lambda i: (0, i)),
            pl.BlockSpec((K, D), lambda i: (0, 0)),
            pl.BlockSpec((K, 1), lambda i: (0, 0)),
            pl.BlockSpec((1, 1, BLOCK), lambda i: (i, 0, 0)),
        ],
        out_specs=[
            pl.BlockSpec((1, 1, BLOCK), lambda i: (i, 0, 0)),
            pl.BlockSpec((1, 1, BLOCK), lambda i: (i, 0, 0)),
            pl.BlockSpec((K, 128), lambda i: (0, 0)),
        ],
        out_shape=[
            jax.ShapeDtypeStruct((nb, 1, BLOCK), jnp.int32),
            jax.ShapeDtypeStruct((nb, 1, BLOCK), jnp.float32),
            jax.ShapeDtypeStruct((K, 128), jnp.float32),
        ],
        compiler_params=pltpu.CompilerParams(
            dimension_semantics=("arbitrary",),
        ),
    )(xt, emb, e2, x2)
    return c.reshape(n), l.reshape(n), table


CHUNK = 128


def _make_gather(n):
    info = plsc.get_sparse_core_info()
    nw = info.num_cores * info.num_subcores
    b_per_w = n // nw
    n_chunks = b_per_w // CHUNK
    mesh = plsc.VectorSubcoreMesh(core_axis_name="c", subcore_axis_name="s")

    @functools.partial(
        pl.kernel, mesh=mesh,
        out_type=jax.ShapeDtypeStruct((n, 128), jnp.float32),
        scratch_types=[
            pltpu.VMEM((n_chunks, CHUNK), jnp.int32),
            pltpu.VMEM((CHUNK, 128), jnp.float32),
            pltpu.VMEM((CHUNK, 128), jnp.float32),
            pltpu.SemaphoreType.DMA,
            pltpu.SemaphoreType.DMA,
            pltpu.SemaphoreType.DMA,
        ],
    )
    def gather(table_hbm, idx_hbm, out_hbm, idx_v, rows_a, rows_b, sem_g, sem_a, sem_b):
        wid = lax.axis_index("s") * info.num_cores + lax.axis_index("c")
        row0 = wid * n_chunks
        pltpu.sync_copy(idx_hbm.at[pl.ds(row0, n_chunks)], idx_v)
        bufs = (rows_a, rows_b)
        sems = (sem_a, sem_b)
        for j in range(n_chunks):
            buf, sem = bufs[j % 2], sems[j % 2]
            if j >= 2:
                pltpu.make_async_copy(
                    buf,
                    out_hbm.at[pl.ds((row0 + j - 2) * CHUNK, CHUNK)],
                    sem).wait()
            pltpu.async_copy(table_hbm.at[idx_v.at[j]], buf, sem_g).wait()
            pltpu.async_copy(buf,
                             out_hbm.at[pl.ds((row0 + j) * CHUNK, CHUNK)],
                             sem)
        for j in range(n_chunks - 2, n_chunks):
            pltpu.make_async_copy(
                bufs[j % 2],
                out_hbm.at[pl.ds((row0 + j) * CHUNK, CHUNK)],
                sems[j % 2]).wait()

    return gather


def kernel(h, emb):
    flat = h.reshape(-1, D)
    n = flat.shape[0]
    c, l, table = _argmin_stage(flat, emb)
    q = _make_gather(n)(table, c.reshape(n // CHUNK, CHUNK))[:, :D]
    return q.reshape(h.shape), c.reshape(n, 1), l


# speedup vs baseline: 1.8382x; 1.0523x over previous
import functools

import jax
import jax.numpy as jnp
from jax import lax
from jax.experimental import pallas as pl
from jax.experimental.pallas import tpu as pltpu
from jax.experimental.pallas import tpu_sc as plsc

K = 1024
D = 32
BLOCK = 2048


def _vq_argmin_kernel(xt_ref, emb_ref, e2_ref, x2_ref, c_ref, l_ref, pad_ref):
    xt = xt_ref[...]
    e = emb_ref[...]
    e2c = e2_ref[...]
    x2r = x2_ref[...].reshape(1, BLOCK)
    xe2 = lax.dot_general(e, xt * -2.0, (((1,), (0,)), ((), ())),
                          preferred_element_type=jnp.float32)
    d = (x2r + e2c) + xe2
    m = jnp.min(d, axis=0, keepdims=True)
    iota = lax.broadcasted_iota(jnp.int32, (K, BLOCK), 0)
    c = jnp.min(jnp.where(d <= m, iota, K), axis=0).astype(jnp.int32)
    c_ref[...] = c.reshape(1, 1, BLOCK)
    l_ref[...] = (m * (0.2 / D)).reshape(1, 1, BLOCK)

    @pl.when(pl.program_id(0) == 0)
    def _():
        pad_ref[...] = jnp.pad(e, ((0, 0), (0, 128 - D)))


def _argmin_stage(flat, emb):
    n = flat.shape[0]
    nb = n // BLOCK
    xt = flat.T
    e2 = jnp.sum(emb ** 2, axis=-1)[:, None]
    x2 = jnp.sum(flat ** 2, axis=-1).reshape(nb, 1, BLOCK)
    c, l, table = pl.pallas_call(
        _vq_argmin_kernel,
        grid=(nb,),
        in_specs=[
            pl.BlockSpec((D, BLOCK), lambda i: (0, i)),
            pl.BlockSpec((K, D), lambda i: (0, 0)),
            pl.BlockSpec((K, 1), lambda i: (0, 0)),
            pl.BlockSpec((1, 1, BLOCK), lambda i: (i, 0, 0)),
        ],
        out_specs=[
            pl.BlockSpec((1, 1, BLOCK), lambda i: (i, 0, 0)),
            pl.BlockSpec((1, 1, BLOCK), lambda i: (i, 0, 0)),
            pl.BlockSpec((K, 128), lambda i: (0, 0)),
        ],
        out_shape=[
            jax.ShapeDtypeStruct((nb, 1, BLOCK), jnp.int32),
            jax.ShapeDtypeStruct((nb, 1, BLOCK), jnp.float32),
            jax.ShapeDtypeStruct((K, 128), jnp.float32),
        ],
        compiler_params=pltpu.CompilerParams(
            dimension_semantics=("arbitrary",),
        ),
    )(xt, emb, e2, x2)
    return c.reshape(n), l.reshape(n), table


CHUNK = 128


def _make_gather(n):
    info = plsc.get_sparse_core_info()
    nw = info.num_cores * info.num_subcores
    b_per_w = n // nw
    n_chunks = b_per_w // CHUNK
    mesh = plsc.VectorSubcoreMesh(core_axis_name="c", subcore_axis_name="s")

    @functools.partial(
        pl.kernel, mesh=mesh,
        out_type=jax.ShapeDtypeStruct((n, 128), jnp.float32),
        scratch_types=[
            pltpu.VMEM((n_chunks, CHUNK), jnp.int32),
            pltpu.VMEM((CHUNK, 128), jnp.float32),
            pltpu.VMEM((CHUNK, 128), jnp.float32),
            pltpu.SemaphoreType.DMA,
            pltpu.SemaphoreType.DMA,
            pltpu.SemaphoreType.DMA,
        ],
    )
    def gather(table_hbm, idx_hbm, out_hbm, idx_v, rows_a, rows_b, sem_g, sem_a, sem_b):
        wid = lax.axis_index("s") * info.num_cores + lax.axis_index("c")
        row0 = wid * n_chunks
        pltpu.sync_copy(idx_hbm.at[pl.ds(row0, n_chunks)], idx_v)
        bufs = (rows_a, rows_b)
        sems = (sem_a, sem_b)
        for j in range(n_chunks):
            buf, sem = bufs[j % 2], sems[j % 2]
            if j >= 2:
                pltpu.make_async_copy(
                    buf,
                    out_hbm.at[pl.ds((row0 + j - 2) * CHUNK, CHUNK)],
                    sem).wait()
            pltpu.async_copy(table_hbm.at[idx_v.at[j]], buf, sem_g).wait()
            pltpu.async_copy(buf,
                             out_hbm.at[pl.ds((row0 + j) * CHUNK, CHUNK)],
                             sem)
        for j in range(n_chunks - 2, n_chunks):
            pltpu.make_async_copy(
                bufs[j % 2],
                out_hbm.at[pl.ds((row0 + j) * CHUNK, CHUNK)],
                sems[j % 2]).wait()

    return gather


def kernel(h, emb):
    flat = h.reshape(-1, D)
    n = flat.shape[0]
    c, l, table = _argmin_stage(flat, emb)
    q = _make_gather(n)(table, c.reshape(n // CHUNK, CHUNK))[:, :D]
    return q.reshape(h.shape), c.reshape(n, 1), l


# speedup vs baseline: 1.8656x; 1.0149x over previous
import functools

import jax
import jax.numpy as jnp
from jax import lax
from jax.experimental import pallas as pl
from jax.experimental.pallas import tpu as pltpu
from jax.experimental.pallas import tpu_sc as plsc

K = 1024
D = 32
BLOCK = 4096


def _vq_argmin_kernel(xt_ref, emb_ref, e2_ref, x2_ref, c_ref, l_ref, pad_ref):
    xt = xt_ref[...]
    e = emb_ref[...]
    e2c = e2_ref[...]
    x2r = x2_ref[...].reshape(1, BLOCK)
    xe2 = lax.dot_general(e, xt * -2.0, (((1,), (0,)), ((), ())),
                          preferred_element_type=jnp.float32)
    d = (x2r + e2c) + xe2
    m = jnp.min(d, axis=0, keepdims=True)
    iota = lax.broadcasted_iota(jnp.int32, (K, BLOCK), 0)
    c = jnp.min(jnp.where(d <= m, iota, K), axis=0).astype(jnp.int32)
    c_ref[...] = c.reshape(1, 1, BLOCK)
    l_ref[...] = (m * (0.2 / D)).reshape(1, 1, BLOCK)

    @pl.when(pl.program_id(0) == 0)
    def _():
        pad_ref[...] = jnp.pad(e, ((0, 0), (0, 128 - D)))


def _argmin_stage(flat, emb):
    n = flat.shape[0]
    nb = n // BLOCK
    xt = flat.T
    e2 = jnp.sum(emb ** 2, axis=-1)[:, None]
    x2 = jnp.sum(flat ** 2, axis=-1).reshape(nb, 1, BLOCK)
    c, l, table = pl.pallas_call(
        _vq_argmin_kernel,
        grid=(nb,),
        in_specs=[
            pl.BlockSpec((D, BLOCK), lambda i: (0, i)),
            pl.BlockSpec((K, D), lambda i: (0, 0)),
            pl.BlockSpec((K, 1), lambda i: (0, 0)),
            pl.BlockSpec((1, 1, BLOCK), lambda i: (i, 0, 0)),
        ],
        out_specs=[
            pl.BlockSpec((1, 1, BLOCK), lambda i: (i, 0, 0)),
            pl.BlockSpec((1, 1, BLOCK), lambda i: (i, 0, 0)),
            pl.BlockSpec((K, 128), lambda i: (0, 0)),
        ],
        out_shape=[
            jax.ShapeDtypeStruct((nb, 1, BLOCK), jnp.int32),
            jax.ShapeDtypeStruct((nb, 1, BLOCK), jnp.float32),
            jax.ShapeDtypeStruct((K, 128), jnp.float32),
        ],
        compiler_params=pltpu.CompilerParams(
            dimension_semantics=("arbitrary",),
        ),
    )(xt, emb, e2, x2)
    return c.reshape(n), l.reshape(n), table


CHUNK = 128


def _make_gather(n):
    info = plsc.get_sparse_core_info()
    nw = info.num_cores * info.num_subcores
    b_per_w = n // nw
    n_chunks = b_per_w // CHUNK
    mesh = plsc.VectorSubcoreMesh(core_axis_name="c", subcore_axis_name="s")

    @functools.partial(
        pl.kernel, mesh=mesh,
        out_type=jax.ShapeDtypeStruct((n, 128), jnp.float32),
        scratch_types=[
            pltpu.VMEM((n_chunks, CHUNK), jnp.int32),
            pltpu.VMEM((CHUNK, 128), jnp.float32),
            pltpu.VMEM((CHUNK, 128), jnp.float32),
            pltpu.SemaphoreType.DMA,
            pltpu.SemaphoreType.DMA,
            pltpu.SemaphoreType.DMA,
        ],
    )
    def gather(table_hbm, idx_hbm, out_hbm, idx_v, rows_a, rows_b, sem_g, sem_a, sem_b):
        wid = lax.axis_index("s") * info.num_cores + lax.axis_index("c")
        row0 = wid * n_chunks
        pltpu.sync_copy(idx_hbm.at[pl.ds(row0, n_chunks)], idx_v)
        bufs = (rows_a, rows_b)
        sems = (sem_a, sem_b)
        for j in range(n_chunks):
            buf, sem = bufs[j % 2], sems[j % 2]
            if j >= 2:
                pltpu.make_async_copy(
                    buf,
                    out_hbm.at[pl.ds((row0 + j - 2) * CHUNK, CHUNK)],
                    sem).wait()
            pltpu.async_copy(table_hbm.at[idx_v.at[j]], buf, sem_g).wait()
            pltpu.async_copy(buf,
                             out_hbm.at[pl.ds((row0 + j) * CHUNK, CHUNK)],
                             sem)
        for j in range(n_chunks - 2, n_chunks):
            pltpu.make_async_copy(
                bufs[j % 2],
                out_hbm.at[pl.ds((row0 + j) * CHUNK, CHUNK)],
                sems[j % 2]).wait()

    return gather


def kernel(h, emb):
    flat = h.reshape(-1, D)
    n = flat.shape[0]
    c, l, table = _argmin_stage(flat, emb)
    q = _make_gather(n)(table, c.reshape(n // CHUNK, CHUNK))[:, :D]
    return q.reshape(h.shape), c.reshape(n, 1), l


# speedup vs baseline: 1.8698x; 1.0023x over previous
import functools

import jax
import jax.numpy as jnp
from jax import lax
from jax.experimental import pallas as pl
from jax.experimental.pallas import tpu as pltpu
from jax.experimental.pallas import tpu_sc as plsc

K = 1024
D = 32
BLOCK = 4096


def _vq_argmin_kernel(xt_ref, emb_ref, e2_ref, x2_ref, c_ref, l_ref, pad_ref):
    xt = xt_ref[...]
    e = emb_ref[...]
    e2c = e2_ref[...]
    x2r = x2_ref[...].reshape(1, BLOCK)
    xe2 = lax.dot_general(e, xt * -2.0, (((1,), (0,)), ((), ())),
                          preferred_element_type=jnp.float32)
    d = (x2r + e2c) + xe2
    m = jnp.min(d, axis=0, keepdims=True)
    iota = lax.broadcasted_iota(jnp.int32, (K, BLOCK), 0)
    c = jnp.min(jnp.where(d <= m, iota, K), axis=0).astype(jnp.int32)
    c_ref[...] = c.reshape(1, 1, BLOCK)
    l_ref[...] = (m * (0.2 / D)).reshape(1, 1, BLOCK)

    @pl.when(pl.program_id(0) == 0)
    def _():
        pad_ref[...] = jnp.pad(e, ((0, 0), (0, 128 - D)))


def _argmin_stage(flat, emb):
    n = flat.shape[0]
    nb = n // BLOCK
    xt = flat.T
    e2 = jnp.sum(emb ** 2, axis=-1)[:, None]
    x2 = jnp.sum(flat ** 2, axis=-1).reshape(nb, 1, BLOCK)
    c, l, table = pl.pallas_call(
        _vq_argmin_kernel,
        grid=(nb,),
        in_specs=[
            pl.BlockSpec((D, BLOCK), lambda i: (0, i)),
            pl.BlockSpec((K, D), lambda i: (0, 0)),
            pl.BlockSpec((K, 1), lambda i: (0, 0)),
            pl.BlockSpec((1, 1, BLOCK), lambda i: (i, 0, 0)),
        ],
        out_specs=[
            pl.BlockSpec((1, 1, BLOCK), lambda i: (i, 0, 0)),
            pl.BlockSpec((1, 1, BLOCK), lambda i: (i, 0, 0)),
            pl.BlockSpec((K, 128), lambda i: (0, 0)),
        ],
        out_shape=[
            jax.ShapeDtypeStruct((nb, 1, BLOCK), jnp.int32),
            jax.ShapeDtypeStruct((nb, 1, BLOCK), jnp.float32),
            jax.ShapeDtypeStruct((K, 128), jnp.float32),
        ],
        compiler_params=pltpu.CompilerParams(
            dimension_semantics=("parallel",),
        ),
    )(xt, emb, e2, x2)
    return c.reshape(n), l.reshape(n), table


CHUNK = 128


def _make_gather(n):
    info = plsc.get_sparse_core_info()
    nw = info.num_cores * info.num_subcores
    b_per_w = n // nw
    n_chunks = b_per_w // CHUNK
    mesh = plsc.VectorSubcoreMesh(core_axis_name="c", subcore_axis_name="s")

    @functools.partial(
        pl.kernel, mesh=mesh,
        out_type=jax.ShapeDtypeStruct((n, 128), jnp.float32),
        scratch_types=[
            pltpu.VMEM((n_chunks, CHUNK), jnp.int32),
            pltpu.VMEM((CHUNK, 128), jnp.float32),
            pltpu.VMEM((CHUNK, 128), jnp.float32),
            pltpu.SemaphoreType.DMA,
            pltpu.SemaphoreType.DMA,
            pltpu.SemaphoreType.DMA,
        ],
    )
    def gather(table_hbm, idx_hbm, out_hbm, idx_v, rows_a, rows_b, sem_g, sem_a, sem_b):
        wid = lax.axis_index("s") * info.num_cores + lax.axis_index("c")
        row0 = wid * n_chunks
        pltpu.sync_copy(idx_hbm.at[pl.ds(row0, n_chunks)], idx_v)
        bufs = (rows_a, rows_b)
        sems = (sem_a, sem_b)
        for j in range(n_chunks):
            buf, sem = bufs[j % 2], sems[j % 2]
            if j >= 2:
                pltpu.make_async_copy(
                    buf,
                    out_hbm.at[pl.ds((row0 + j - 2) * CHUNK, CHUNK)],
                    sem).wait()
            pltpu.async_copy(table_hbm.at[idx_v.at[j]], buf, sem_g).wait()
            pltpu.async_copy(buf,
                             out_hbm.at[pl.ds((row0 + j) * CHUNK, CHUNK)],
                             sem)
        for j in range(n_chunks - 2, n_chunks):
            pltpu.make_async_copy(
                bufs[j % 2],
                out_hbm.at[pl.ds((row0 + j) * CHUNK, CHUNK)],
                sems[j % 2]).wait()

    return gather


def kernel(h, emb):
    flat = h.reshape(-1, D)
    n = flat.shape[0]
    c, l, table = _argmin_stage(flat, emb)
    q = _make_gather(n)(table, c.reshape(n // CHUNK, CHUNK))[:, :D]
    return q.reshape(h.shape), c.reshape(n, 1), l
